# Initial kernel scaffold; baseline (speedup 1.0000x reference)
import jax
import jax.numpy as jnp
from jax.experimental import pallas as pl


def kernel(x_omic1, x_omic2, edge_index, W1a, b1a, W2a, b2a, W1b, b1b, W2b, b2b, Wfc, bfc):
    raise NotImplementedError("write your pallas kernel here")


# speedup vs baseline: 9.4601x; 9.4601x over previous
import functools

import jax
import jax.numpy as jnp
from jax import lax
from jax.experimental import pallas as pl
from jax.experimental.pallas import tpu as pltpu
from jax.experimental.pallas import tpu_sc as plsc

_N = 50000
_E = 800000
_HALF = 25000
_HALF_PAD = 25088
_NP = 2 * _HALF_PAD
_NS = 16
_EB = 128
_KB = 2
_TB = 196
_NBLK = _KB * _TB
_EP = _NS * _NBLK * _EB
_ZR = _HALF_PAD // _NS


def _make_edge_pass(fw, gather):
    mesh = plsc.VectorSubcoreMesh(core_axis_name="c", subcore_axis_name="s")
    scratch = [
        pltpu.VMEM_SHARED((_HALF_PAD, fw), jnp.float32),
        pltpu.VMEM((_KB * _EB, fw), jnp.float32),
        pltpu.VMEM((_KB, _EB), jnp.int32),
        pltpu.VMEM((_KB, _EB), jnp.int32),
        pltpu.SemaphoreType.DMA,
    ]
    if gather:
        scratch += [
            pltpu.VMEM((_KB, _EB), jnp.int32),
            pltpu.VMEM((_KB, _EB), jnp.int32),
        ]

    def body(*refs):
        if gather:
            z_hbm, src_hbm, dst_hbm, out_hbm, acc, rows, draw, didx, sem, sraw, sidx = refs
        else:
            dst_hbm, out_hbm, acc, rows, draw, didx, sem = refs
        c = lax.axis_index("c")
        s = lax.axis_index("s")
        base = c * _HALF
        nrows = _KB * _EB

        def fill(val):
            def fill_i(i, carry):
                for h in range(fw // 16):
                    rows[i, pl.ds(h * 16, 16)] = jnp.full((16,), val, jnp.float32)
                return carry
            lax.fori_loop(0, nrows, fill_i, 0)

        fill(0.0)
        nfull, rem = _ZR // nrows, _ZR % nrows
        for q in range(nfull):
            pltpu.sync_copy(rows, acc.at[pl.ds(s * _ZR + q * nrows, nrows)])
        if rem:
            pltpu.sync_copy(rows.at[pl.ds(0, rem)],
                            acc.at[pl.ds(s * _ZR + nfull * nrows, rem)])
        if not gather:
            fill(1.0)
        plsc.subcore_barrier()

        it = lax.iota(jnp.int32, 16)

        def step(t, carry):
            pltpu.sync_copy(dst_hbm.at[s, pl.ds(t * _KB, _KB)], draw)
            if gather:
                pltpu.sync_copy(src_hbm.at[s, pl.ds(t * _KB, _KB)], sraw)
            for j in range(_KB):
                def chunk(q, c2):
                    dv = draw[j, pl.ds(q * 16, 16)]
                    lv = dv - base
                    ok = (lv >= 0) & (lv < _HALF)
                    trash = (_HALF + 24) + it + (q & 3) * 16
                    didx[j, pl.ds(q * 16, 16)] = jnp.where(ok, lv, trash)
                    if gather:
                        sv = sraw[j, pl.ds(q * 16, 16)]
                        sidx[j, pl.ds(q * 16, 16)] = jnp.where(
                            sv >= _HALF, sv + (_HALF_PAD - _HALF), sv)
                    return c2
                lax.fori_loop(0, _EB // 16, chunk, 0)
            if gather:
                cps = [pltpu.async_copy(z_hbm.at[sidx.at[j]],
                                        rows.at[pl.ds(j * _EB, _EB)], sem)
                       for j in range(_KB)]
                for cp in cps:
                    cp.wait()
            for j in range(_KB):
                pltpu.sync_copy(rows.at[pl.ds(j * _EB, _EB)],
                                acc.at[didx.at[j]], add=True)
            return carry
        lax.fori_loop(0, _TB, step, 0)

        plsc.subcore_barrier()
        pltpu.sync_copy(acc.at[pl.ds(s * _ZR, _ZR)],
                        out_hbm.at[pl.ds(c * _HALF_PAD + s * _ZR, _ZR)])

    return pl.kernel(
        body,
        out_type=jax.ShapeDtypeStruct((_NP, fw), jnp.float32),
        mesh=mesh,
        scratch_types=scratch,
        compiler_params=pltpu.CompilerParams(use_tc_tiling_on_sc=False),
    )


_seg_sum = _make_edge_pass(64, True)
_deg_count = _make_edge_pass(16, False)

_BM = 1024


def _mm_scale(x, w, deg16):
    np_, d = x.shape
    h = w.shape[1]

    def body(x_ref, w_ref, g_ref, o_ref):
        dinv = lax.rsqrt(g_ref[:, 0:1] + 1.0)
        o_ref[...] = dinv * jnp.dot(x_ref[...], w_ref[...],
                                    preferred_element_type=jnp.float32)

    return pl.pallas_call(
        body,
        grid=(np_ // _BM,),
        in_specs=[
            pl.BlockSpec((_BM, d), lambda i: (i, 0)),
            pl.BlockSpec((d, h), lambda i: (0, 0)),
            pl.BlockSpec((_BM, 16), lambda i: (i, 0)),
        ],
        out_specs=pl.BlockSpec((_BM, h), lambda i: (i, 0)),
        out_shape=jax.ShapeDtypeStruct((np_, h), jnp.float32),
    )(x, w, deg16)


def _post_elu(y2, z, deg16, b):
    np_, h = z.shape

    def body(y_ref, z_ref, g_ref, b_ref, o_ref):
        dinv = lax.rsqrt(g_ref[:, 0:1] + 1.0)
        v = dinv * (y_ref[...] + z_ref[...]) + b_ref[...]
        o_ref[...] = jnp.where(v > 0, v, jnp.exp(v) - 1.0)

    return pl.pallas_call(
        body,
        grid=(np_ // _BM,),
        in_specs=[
            pl.BlockSpec((_BM, h), lambda i: (i, 0)),
            pl.BlockSpec((_BM, h), lambda i: (i, 0)),
            pl.BlockSpec((_BM, 16), lambda i: (i, 0)),
            pl.BlockSpec((1, h), lambda i: (0, 0)),
        ],
        out_specs=pl.BlockSpec((_BM, h), lambda i: (i, 0)),
        out_shape=jax.ShapeDtypeStruct((np_, h), jnp.float32),
    )(y2, z, deg16, b)


def _final_fc(x1, x2, wfc, bfc):
    np_, h = x1.shape
    co = wfc.shape[1]

    def body(a_ref, b2_ref, w_ref, c_ref, o_ref):
        o_ref[...] = (jnp.dot(a_ref[...], w_ref[0:h, :],
                              preferred_element_type=jnp.float32)
                      + jnp.dot(b2_ref[...], w_ref[h:2 * h, :],
                                preferred_element_type=jnp.float32)
                      + c_ref[...])

    return pl.pallas_call(
        body,
        grid=(np_ // _BM,),
        in_specs=[
            pl.BlockSpec((_BM, h), lambda i: (i, 0)),
            pl.BlockSpec((_BM, h), lambda i: (i, 0)),
            pl.BlockSpec((2 * h, co), lambda i: (0, 0)),
            pl.BlockSpec((1, co), lambda i: (0, 0)),
        ],
        out_specs=pl.BlockSpec((_BM, co), lambda i: (i, 0)),
        out_shape=jax.ShapeDtypeStruct((np_, co), jnp.float32),
    )(x1, x2, wfc, bfc)


def _pad_half(x):
    d = x.shape[1]
    pad = jnp.zeros((_HALF_PAD - _HALF, d), x.dtype)
    return jnp.concatenate([x[:_HALF], pad, x[_HALF:], pad], axis=0)


def kernel(x_omic1, x_omic2, edge_index, W1a, b1a, W2a, b2a, W1b, b1b, W2b, b2b, Wfc, bfc):
    src = edge_index[0]
    dst = edge_index[1]
    padn = _EP - _E
    srcm = jnp.concatenate([src, jnp.zeros((padn,), jnp.int32)]
                           ).reshape(_NS, _NBLK, _EB)
    dstm = jnp.concatenate([dst, jnp.full((padn,), _N, jnp.int32)]
                           ).reshape(_NS, _NBLK, _EB)

    deg16 = _deg_count(dstm)

    xp1 = _pad_half(x_omic1)
    xp2 = _pad_half(x_omic2)

    def branch(xp, w1, bb1, w2, bb2):
        z = _mm_scale(xp, w1, deg16)
        y2 = _seg_sum(z, srcm, dstm)
        h1 = _post_elu(y2, z, deg16, bb1.reshape(1, -1))
        z2 = _mm_scale(h1, w2, deg16)
        y22 = _seg_sum(z2, srcm, dstm)
        return _post_elu(y22, z2, deg16, bb2.reshape(1, -1))

    h1 = branch(xp1, W1a, b1a, W2a, b2a)
    h2 = branch(xp2, W1b, b1b, W2b, b2b)
    outp = _final_fc(h1, h2, Wfc, bfc.reshape(1, -1))
    return jnp.concatenate([outp[:_HALF], outp[_HALF_PAD:_HALF_PAD + _HALF]],
                           axis=0)


# speedup vs baseline: 13.6938x; 1.4475x over previous
import functools

import jax
import jax.numpy as jnp
from jax import lax
from jax.experimental import pallas as pl
from jax.experimental.pallas import tpu as pltpu
from jax.experimental.pallas import tpu_sc as plsc

_N = 50000
_E = 800000
_HALF = 25000
_HALF_PAD = 25088
_NP = 2 * _HALF_PAD
_NS = 16
_EB = 128
_SB = 8
_NSB = 49
_NBLK = _SB * _NSB
_EP = _NS * _NBLK * _EB
_ZR = _HALF_PAD // _NS


def _make_edge_pass(fw, gather):
    mesh = plsc.VectorSubcoreMesh(core_axis_name="c", subcore_axis_name="s")
    scratch = [
        pltpu.VMEM_SHARED((_HALF_PAD, fw), jnp.float32),
        pltpu.VMEM((_EB, fw), jnp.float32),
        pltpu.VMEM((1, _EB), jnp.int32),
        pltpu.VMEM((1, _EB), jnp.int32),
        pltpu.VMEM((_SB, _EB), jnp.int32),
        pltpu.SemaphoreType.DMA,
        pltpu.SemaphoreType.DMA,
    ]
    if gather:
        scratch += [
            pltpu.VMEM((_EB, fw), jnp.float32),
            pltpu.VMEM((1, _EB), jnp.int32),
            pltpu.VMEM((1, _EB), jnp.int32),
            pltpu.VMEM((_SB, _EB), jnp.int32),
        ]

    def body(*refs):
        if gather:
            (z_hbm, src_hbm, dst_hbm, out_hbm, acc, rows_a, didx_a, didx_b,
             dstage, sem_a, sem_b, rows_b, sidx_a, sidx_b, sstage) = refs
        else:
            (dst_hbm, out_hbm, acc, rows_a, didx_a, didx_b, dstage,
             sem_a, sem_b) = refs
            rows_b = rows_a
        c = lax.axis_index("c")
        s = lax.axis_index("s")
        base = c * _HALF
        it = lax.iota(jnp.int32, 16)
        rows_sets = (rows_a, rows_b)
        didx_sets = (didx_a, didx_b)
        sems = (sem_a, sem_b)
        sidx_sets = (sidx_a, sidx_b) if gather else (None, None)

        def fill(val):
            def fill_i(i, carry):
                for h in range(fw // 16):
                    rows_a[i, pl.ds(h * 16, 16)] = jnp.full((16,), val,
                                                            jnp.float32)
                return carry
            lax.fori_loop(0, _EB, fill_i, 0)

        fill(0.0)
        nfull, rem = _ZR // _EB, _ZR % _EB
        for q in range(nfull):
            pltpu.sync_copy(rows_a, acc.at[pl.ds(s * _ZR + q * _EB, _EB)])
        if rem:
            pltpu.sync_copy(rows_a.at[pl.ds(0, rem)],
                            acc.at[pl.ds(s * _ZR + nfull * _EB, rem)])
        if not gather:
            fill(1.0)
        plsc.subcore_barrier()

        def load_stage(sb):
            pltpu.sync_copy(dst_hbm.at[s, pl.ds(sb * _SB, _SB)], dstage)
            if gather:
                pltpu.sync_copy(src_hbm.at[s, pl.ds(sb * _SB, _SB)], sstage)

        def compute_idx(r, x):
            def chunk(q, c2):
                dv = dstage[r, pl.ds(q * 16, 16)]
                lv = dv - base
                ok = (lv >= 0) & (lv < _HALF)
                trash = (_HALF + 24) + it + (q & 3) * 16
                didx_sets[x][0, pl.ds(q * 16, 16)] = jnp.where(ok, lv, trash)
                if gather:
                    sv = sstage[r, pl.ds(q * 16, 16)]
                    sidx_sets[x][0, pl.ds(q * 16, 16)] = jnp.where(
                        sv >= _HALF, sv + (_HALF_PAD - _HALF), sv)
                return c2
            lax.fori_loop(0, _EB // 16, chunk, 0)

        def gather_start(x):
            pltpu.async_copy(z_hbm.at[sidx_sets[x].at[0]], rows_sets[x],
                             sems[x])

        def gather_wait(x):
            pltpu.make_async_copy(z_hbm.at[sidx_sets[x].at[0]], rows_sets[x],
                                  sems[x]).wait()

        def scatter_start(x):
            pltpu.async_copy(rows_sets[x], acc.at[didx_sets[x].at[0]],
                             sems[x], add=True)

        def scatter_wait(x):
            pltpu.make_async_copy(rows_sets[x], acc.at[didx_sets[x].at[0]],
                                  sems[x]).wait()

        if gather:
            def start(x):
                gather_start(x)

            def finish(x):
                gather_wait(x)
                scatter_start(x)
                scatter_wait(x)
        else:
            start = scatter_start
            finish = scatter_wait

        load_stage(0)
        compute_idx(0, 0)
        start(0)
        compute_idx(1, 1)
        start(1)

        def super_body(sb, carry):
            for j in range(_SB):
                x = j & 1
                finish(x)
                if j < _SB - 2:
                    compute_idx(j + 2, x)
                else:
                    if j == _SB - 2:
                        load_stage(sb + 1)
                    compute_idx(j + 2 - _SB, x)
                start(x)
            return carry
        lax.fori_loop(0, _NSB - 1, super_body, 0)

        for j in range(_SB):
            x = j & 1
            finish(x)
            if j < _SB - 2:
                compute_idx(j + 2, x)
                start(x)

        plsc.subcore_barrier()
        pltpu.sync_copy(acc.at[pl.ds(s * _ZR, _ZR)],
                        out_hbm.at[pl.ds(c * _HALF_PAD + s * _ZR, _ZR)])

    return pl.kernel(
        body,
        out_type=jax.ShapeDtypeStruct((_NP, fw), jnp.float32),
        mesh=mesh,
        scratch_types=scratch,
        compiler_params=pltpu.CompilerParams(use_tc_tiling_on_sc=False),
    )


_seg_sum = _make_edge_pass(64, True)
_deg_count = _make_edge_pass(16, False)

_BM = 1024


def _mm_scale(x, w, deg16):
    np_, d = x.shape
    h = w.shape[1]

    def body(x_ref, w_ref, g_ref, o_ref):
        dinv = lax.rsqrt(g_ref[:, 0:1] + 1.0)
        o_ref[...] = dinv * jnp.dot(x_ref[...], w_ref[...],
                                    preferred_element_type=jnp.float32)

    return pl.pallas_call(
        body,
        grid=(np_ // _BM,),
        in_specs=[
            pl.BlockSpec((_BM, d), lambda i: (i, 0)),
            pl.BlockSpec((d, h), lambda i: (0, 0)),
            pl.BlockSpec((_BM, 16), lambda i: (i, 0)),
        ],
        out_specs=pl.BlockSpec((_BM, h), lambda i: (i, 0)),
        out_shape=jax.ShapeDtypeStruct((np_, h), jnp.float32),
    )(x, w, deg16)


def _post_elu(y2, z, deg16, b):
    np_, h = z.shape

    def body(y_ref, z_ref, g_ref, b_ref, o_ref):
        dinv = lax.rsqrt(g_ref[:, 0:1] + 1.0)
        v = dinv * (y_ref[...] + z_ref[...]) + b_ref[...]
        o_ref[...] = jnp.where(v > 0, v, jnp.exp(v) - 1.0)

    return pl.pallas_call(
        body,
        grid=(np_ // _BM,),
        in_specs=[
            pl.BlockSpec((_BM, h), lambda i: (i, 0)),
            pl.BlockSpec((_BM, h), lambda i: (i, 0)),
            pl.BlockSpec((_BM, 16), lambda i: (i, 0)),
            pl.BlockSpec((1, h), lambda i: (0, 0)),
        ],
        out_specs=pl.BlockSpec((_BM, h), lambda i: (i, 0)),
        out_shape=jax.ShapeDtypeStruct((np_, h), jnp.float32),
    )(y2, z, deg16, b)


def _final_fc(x1, x2, wfc, bfc):
    np_, h = x1.shape
    co = wfc.shape[1]

    def body(a_ref, b2_ref, w_ref, c_ref, o_ref):
        o_ref[...] = (jnp.dot(a_ref[...], w_ref[0:h, :],
                              preferred_element_type=jnp.float32)
                      + jnp.dot(b2_ref[...], w_ref[h:2 * h, :],
                                preferred_element_type=jnp.float32)
                      + c_ref[...])

    return pl.pallas_call(
        body,
        grid=(np_ // _BM,),
        in_specs=[
            pl.BlockSpec((_BM, h), lambda i: (i, 0)),
            pl.BlockSpec((_BM, h), lambda i: (i, 0)),
            pl.BlockSpec((2 * h, co), lambda i: (0, 0)),
            pl.BlockSpec((1, co), lambda i: (0, 0)),
        ],
        out_specs=pl.BlockSpec((_BM, co), lambda i: (i, 0)),
        out_shape=jax.ShapeDtypeStruct((np_, co), jnp.float32),
    )(x1, x2, wfc, bfc)


def _pad_half(x):
    d = x.shape[1]
    pad = jnp.zeros((_HALF_PAD - _HALF, d), x.dtype)
    return jnp.concatenate([x[:_HALF], pad, x[_HALF:], pad], axis=0)


def kernel(x_omic1, x_omic2, edge_index, W1a, b1a, W2a, b2a, W1b, b1b, W2b, b2b, Wfc, bfc):
    src = edge_index[0]
    dst = edge_index[1]
    padn = _EP - _E
    srcm = jnp.concatenate([src, jnp.zeros((padn,), jnp.int32)]
                           ).reshape(_NS, _NBLK, _EB)
    dstm = jnp.concatenate([dst, jnp.full((padn,), _N, jnp.int32)]
                           ).reshape(_NS, _NBLK, _EB)

    deg16 = _deg_count(dstm)

    xp1 = _pad_half(x_omic1)
    xp2 = _pad_half(x_omic2)

    def branch(xp, w1, bb1, w2, bb2):
        z = _mm_scale(xp, w1, deg16)
        y2 = _seg_sum(z, srcm, dstm)
        h1 = _post_elu(y2, z, deg16, bb1.reshape(1, -1))
        z2 = _mm_scale(h1, w2, deg16)
        y22 = _seg_sum(z2, srcm, dstm)
        return _post_elu(y22, z2, deg16, bb2.reshape(1, -1))

    h1 = branch(xp1, W1a, b1a, W2a, b2a)
    h2 = branch(xp2, W1b, b1b, W2b, b2b)
    outp = _final_fc(h1, h2, Wfc, bfc.reshape(1, -1))
    return jnp.concatenate([outp[:_HALF], outp[_HALF_PAD:_HALF_PAD + _HALF]],
                           axis=0)
